# baseline (speedup 1.0000x reference)
import jax
import jax.numpy as jnp

K = 32
HEADS = 4
HEAD_DIM = 64


def kernel(h_obs, pos_obs, pos_query, W_v, b_v, W1, b1, W2, b2):
    n_o = pos_obs.shape[0]
    n_q = pos_query.shape[0]
    q2 = jnp.sum(pos_query ** 2, axis=1, keepdims=True)
    o2 = jnp.sum(pos_obs ** 2, axis=1)[None, :]
    d2 = q2 + o2 - 2.0 * (pos_query @ pos_obs.T)
    _, knn_idx = jax.lax.top_k(-d2, K)
    src = knn_idx.reshape(-1)
    dst = jnp.repeat(jnp.arange(n_q, dtype=jnp.int32), K)
    rel = pos_query[dst] - pos_obs[src]
    hmid = jax.nn.relu(rel @ W1 + b1)
    logits = hmid @ W2 + b2
    v = (h_obs @ W_v + b_v).reshape(n_o, HEADS, HEAD_DIM)
    v_src = v[src]
    seg_max = jax.ops.segment_max(logits, dst, num_segments=n_q)
    seg_max = jnp.where(jnp.isfinite(seg_max), seg_max, 0.0)
    shifted = logits - seg_max[dst]
    expv = jnp.exp(shifted)
    denom = jax.ops.segment_sum(expv, dst, num_segments=n_q)
    attn = expv / (denom[dst] + 1e-16)
    out = attn[..., None] * v_src
    h_query = jax.ops.segment_sum(out, dst, num_segments=n_q)
    return h_query.reshape(n_q, HEADS * HEAD_DIM)


# speedup vs baseline: 20.0492x; 20.0492x over previous
import functools

import jax
import jax.numpy as jnp
from jax import lax
from jax.experimental import pallas as pl
from jax.experimental.pallas import tpu as pltpu, tpu_sc as plsc

N_O = 10000
N_OP = 10112
N_Q = 4096
LATENT = 256
HEADS = 4
HEAD_DIM = 64
TOTAL_DIM = HEADS * HEAD_DIM
K = 32
E = N_Q * K

QBLK_A = 128
N_CHUNKS = N_OP // 128
TOPT = 5
BIG = 3.0e38

NC = 2
NS = 16
NW = NC * NS
QPW = N_Q // NW
EPW = E // NW
QB_E = 4
NB_E = QPW // QB_E
EB_C = 128
NB_C = EPW // EB_C


def _sel_body(d2_ref, idx_ref):
    keys = d2_ref[...]
    lane128 = lax.broadcasted_iota(jnp.int32, (QBLK_A, 128), 1).astype(jnp.float32)

    m = [jnp.full((QBLK_A, 128), BIG, jnp.float32) for _ in range(TOPT)]
    mi = [jnp.zeros((QBLK_A, 128), jnp.float32) for _ in range(TOPT)]
    for c in range(N_CHUNKS):
        x = keys[:, c * 128:(c + 1) * 128]
        xi = lane128 + float(c * 128)
        for t in range(TOPT):
            cl = x < m[t]
            nm = jnp.minimum(m[t], x)
            nmi = jnp.where(cl, xi, mi[t])
            nx = jnp.maximum(m[t], x)
            nxi = jnp.where(cl, mi[t], xi)
            m[t], mi[t], x, xi = nm, nmi, nx, nxi
    cand = jnp.concatenate(m, axis=1)
    cand_i = jnp.concatenate(mi, axis=1)
    cols = []
    for _ in range(K):
        mn = jnp.min(cand, axis=1, keepdims=True)
        msel = cand == mn
        pick = jnp.min(jnp.where(msel, cand_i, BIG), axis=1, keepdims=True)
        cols.append(pick.astype(jnp.int32))
        cand = jnp.where(msel & (cand_i == pick), BIG, cand)
    idx_ref[...] = jnp.concatenate(cols, axis=1)


_sel_call = pl.pallas_call(
    _sel_body,
    grid=(N_Q // QBLK_A,),
    in_specs=[
        pl.BlockSpec((QBLK_A, N_OP), lambda i: (i, 0)),
    ],
    out_specs=pl.BlockSpec((QBLK_A, K), lambda i: (i, 0)),
    out_shape=jax.ShapeDtypeStruct((N_Q, K), jnp.int32),
)


def _vproj_body(h_ref, wv_ref, bv_ref, v_ref):
    v_ref[...] = lax.dot_general(h_ref[...], wv_ref[...], (((1,), (0,)), ((), ())),
                                 precision=lax.Precision.HIGHEST,
                                 preferred_element_type=jnp.float32) + bv_ref[0:1, :]


RBLK_B = N_OP // 8
_vproj_call = pl.pallas_call(
    _vproj_body,
    grid=(8,),
    in_specs=[
        pl.BlockSpec((RBLK_B, LATENT), lambda i: (i, 0)),
        pl.BlockSpec((LATENT, TOTAL_DIM), lambda i: (0, 0)),
        pl.BlockSpec((8, TOTAL_DIM), lambda i: (0, 0)),
    ],
    out_specs=pl.BlockSpec((RBLK_B, TOTAL_DIM), lambda i: (i, 0)),
    out_shape=jax.ShapeDtypeStruct((N_OP, TOTAL_DIM), jnp.float32),
)


def _pos_gather_body(idx_hbm, pos128_hbm, out_hbm, idx_v, rows_v, cmp_v, sem):
    wid = lax.axis_index("s") * NC + lax.axis_index("c")
    lane = lax.broadcasted_iota(jnp.int32, (16,), 0)

    def body(b, _):
        ebase = wid * EPW + b * EB_C
        pltpu.sync_copy(idx_hbm.at[pl.ds(ebase, EB_C)], idx_v)
        pltpu.async_copy(pos128_hbm.at[idx_v], rows_v, sem).wait()
        for g in range(EB_C // 4):
            acc = jnp.zeros((16,), jnp.float32)
            for j in range(4):
                ve = rows_v[g * 4 + j, pl.ds(0, 16)]
                acc = jnp.where(lane == j * 4, ve[0], acc)
                acc = jnp.where(lane == j * 4 + 1, ve[1], acc)
                acc = jnp.where(lane == j * 4 + 2, ve[2], acc)
            cmp_v[pl.ds(g * 16, 16)] = acc
        pltpu.sync_copy(cmp_v.at[pl.ds(0, EB_C * 4)],
                        out_hbm.at[pl.ds(ebase * 4, EB_C * 4)])
        return 0

    lax.fori_loop(0, NB_C, body, 0)


@functools.lru_cache(maxsize=None)
def _sc_calls():
    mesh = plsc.VectorSubcoreMesh(core_axis_name="c", subcore_axis_name="s",
                                  num_cores=NC, num_subcores=NS)
    pos_gather = pl.kernel(
        _pos_gather_body,
        out_type=jax.ShapeDtypeStruct((E * 4,), jnp.float32),
        mesh=mesh,
        scratch_types=[
            pltpu.VMEM((EB_C,), jnp.int32),
            pltpu.VMEM((EB_C, 128), jnp.float32),
            pltpu.VMEM((EB_C * 4 + 16,), jnp.float32),
            pltpu.SemaphoreType.DMA,
        ],
    )
    wsum = pl.kernel(
        _wsum_body,
        out_type=jax.ShapeDtypeStruct((N_Q, TOTAL_DIM), jnp.float32),
        mesh=mesh,
        scratch_types=[
            pltpu.VMEM((QB_E * K,), jnp.int32),
            pltpu.VMEM((QB_E * K, TOTAL_DIM), jnp.float32),
            pltpu.VMEM((QB_E * K * HEADS + 16,), jnp.float32),
            pltpu.VMEM((QB_E, TOTAL_DIM), jnp.float32),
            pltpu.SemaphoreType.DMA,
        ],
    )
    return pos_gather, wsum


QBLK_D = 128
EBLK_D = QBLK_D * K


def _mlp_body(ps_ref, qp4_ref, w1_ref, w2_ref, b2_ref, attn_ref):
    ps = ps_ref[...].reshape(EBLK_D, 4)
    a = lax.dot_general(ps, w1_ref[...], (((1,), (0,)), ((), ())),
                        precision=lax.Precision.HIGHEST,
                        preferred_element_type=jnp.float32)
    b = lax.dot_general(qp4_ref[...], w1_ref[...], (((1,), (0,)), ((), ())),
                        precision=lax.Precision.HIGHEST,
                        preferred_element_type=jnp.float32)
    bx = jnp.broadcast_to(b[:, None, :], (QBLK_D, K, LATENT)).reshape(EBLK_D, LATENT)
    h = jnp.maximum(bx - a, 0.0)
    logits = lax.dot_general(h, w2_ref[...], (((1,), (0,)), ((), ())),
                             precision=lax.Precision.HIGHEST,
                             preferred_element_type=jnp.float32) + b2_ref[0:1, 0:HEADS]
    l3 = logits.reshape(QBLK_D, K, HEADS)
    mx = jnp.max(l3, axis=1, keepdims=True)
    ex = jnp.exp(l3 - mx)
    sm = jnp.sum(ex, axis=1, keepdims=True)
    attn_ref[...] = ex / (sm + 1e-16)


_mlp_call = pl.pallas_call(
    _mlp_body,
    grid=(N_Q // QBLK_D,),
    in_specs=[
        pl.BlockSpec((QBLK_D, K, 4), lambda i: (i, 0, 0)),
        pl.BlockSpec((QBLK_D, 4), lambda i: (i, 0)),
        pl.BlockSpec((4, LATENT), lambda i: (0, 0)),
        pl.BlockSpec((LATENT, HEADS), lambda i: (0, 0)),
        pl.BlockSpec((8, 128), lambda i: (0, 0)),
    ],
    out_specs=pl.BlockSpec((QBLK_D, K, HEADS), lambda i: (i, 0, 0)),
    out_shape=jax.ShapeDtypeStruct((N_Q, K, HEADS), jnp.float32),
)


def _wsum_body(idx_hbm, attn_hbm, v_hbm, out_hbm, idx_v, rows_v, attn_v, out_v, sem):
    wid = lax.axis_index("s") * NC + lax.axis_index("c")

    def body(b, _):
        ebase = wid * EPW + b * (QB_E * K)
        qbase = wid * QPW + b * QB_E
        pltpu.sync_copy(idx_hbm.at[pl.ds(ebase, QB_E * K)], idx_v)
        cp = pltpu.async_copy(v_hbm.at[idx_v], rows_v, sem)
        pltpu.sync_copy(attn_hbm.at[pl.ds(ebase * HEADS, QB_E * K * HEADS)],
                        attn_v.at[pl.ds(0, QB_E * K * HEADS)])
        cp.wait()
        for qq in range(QB_E):
            def kbody(k, accs):
                new = list(accs)
                row = qq * K + k
                av = attn_v[pl.ds(qq * (K * HEADS) + k * HEADS, 16)]
                for h in range(HEADS):
                    wv = jnp.full((16,), av[h], jnp.float32)
                    for t in range(HEAD_DIM // 16):
                        j = h * 4 + t
                        new[j] = new[j] + wv * rows_v[row, pl.ds(h * HEAD_DIM + t * 16, 16)]
                return tuple(new)

            accs = lax.fori_loop(
                0, K, kbody,
                tuple(jnp.zeros((16,), jnp.float32) for _ in range(16)))
            for h in range(HEADS):
                for t in range(HEAD_DIM // 16):
                    out_v[qq, pl.ds(h * HEAD_DIM + t * 16, 16)] = accs[h * 4 + t]
        pltpu.sync_copy(out_v, out_hbm.at[pl.ds(qbase, QB_E), :])
        return 0

    lax.fori_loop(0, NB_E, body, 0)


def kernel(h_obs, pos_obs, pos_query, W_v, b_v, W1, b1, W2, b2):
    f32 = jnp.float32
    q2 = jnp.sum(pos_query ** 2, axis=1, keepdims=True)
    o2 = jnp.sum(pos_obs ** 2, axis=1)[None, :]
    d2 = q2 + o2 - 2.0 * (pos_query @ pos_obs.T)
    d2p = jnp.concatenate(
        [d2, jnp.full((N_Q, N_OP - N_O), 1.0e30, f32)], axis=1)

    h_pad = jnp.zeros((N_OP, LATENT), f32).at[0:N_O, :].set(h_obs)
    bv_p = jnp.zeros((8, TOTAL_DIM), f32).at[0, :].set(b_v)
    pos128 = jnp.zeros((N_O, 128), f32).at[:, 0:3].set(pos_obs)
    qp4 = jnp.zeros((N_Q, 4), f32).at[:, 0:3].set(pos_query)
    qp4 = qp4.at[:, 3].set(1.0)
    w1p = jnp.zeros((4, LATENT), f32).at[0:3, :].set(W1)
    w1p = w1p.at[3, :].set(b1)
    b2p = jnp.zeros((8, 128), f32).at[0, 0:HEADS].set(b2)

    pos_gather_call, wsum_call = _sc_calls()
    knn_idx = _sel_call(d2p)
    idx_flat = knn_idx.reshape(E)
    v = _vproj_call(h_pad, W_v, bv_p)
    pos_src = pos_gather_call(idx_flat, pos128)
    attn = _mlp_call(pos_src.reshape(N_Q, K, 4), qp4, w1p, W2, b2p)
    attn_flat = attn.reshape(E * HEADS)
    out = wsum_call(idx_flat, attn_flat, v)
    return out


# speedup vs baseline: 23.3611x; 1.1652x over previous
import functools

import jax
import jax.numpy as jnp
from jax import lax
from jax.experimental import pallas as pl
from jax.experimental.pallas import tpu as pltpu, tpu_sc as plsc

N_O = 10000
N_OP = 10112
N_Q = 4096
LATENT = 256
HEADS = 4
HEAD_DIM = 64
TOTAL_DIM = HEADS * HEAD_DIM
K = 32
E = N_Q * K

QBLK_A = 128
N_CHUNKS = N_OP // 128
TOPT = 5
BIG = 3.0e38

NC = 2
NS = 16
NW = NC * NS
QPW = N_Q // NW
EPW = E // NW
QB_E = 4
NB_E = QPW // QB_E
EB_C = 128
NB_C = EPW // EB_C


def _sel_body(d2_ref, idx_ref):
    keys = d2_ref[...]
    lane128 = lax.broadcasted_iota(jnp.int32, (QBLK_A, 128), 1).astype(jnp.float32)

    m = [jnp.full((QBLK_A, 128), BIG, jnp.float32) for _ in range(TOPT)]
    mi = [jnp.zeros((QBLK_A, 128), jnp.float32) for _ in range(TOPT)]
    for c in range(N_CHUNKS):
        x = keys[:, c * 128:(c + 1) * 128]
        xi = lane128 + float(c * 128)
        for t in range(TOPT):
            cl = x < m[t]
            nm = jnp.minimum(m[t], x)
            nmi = jnp.where(cl, xi, mi[t])
            nx = jnp.maximum(m[t], x)
            nxi = jnp.where(cl, mi[t], xi)
            m[t], mi[t], x, xi = nm, nmi, nx, nxi
    cand = jnp.concatenate(m, axis=1)
    cand_i = jnp.concatenate(mi, axis=1)
    cols = []
    for _ in range(K):
        mn = jnp.min(cand, axis=1, keepdims=True)
        msel = cand == mn
        pick = jnp.min(jnp.where(msel, cand_i, BIG), axis=1, keepdims=True)
        cols.append(pick.astype(jnp.int32))
        cand = jnp.where(msel & (cand_i == pick), BIG, cand)
    idx_ref[...] = jnp.concatenate(cols, axis=1)


_sel_call = pl.pallas_call(
    _sel_body,
    grid=(N_Q // QBLK_A,),
    in_specs=[
        pl.BlockSpec((QBLK_A, N_OP), lambda i: (i, 0)),
    ],
    out_specs=pl.BlockSpec((QBLK_A, K), lambda i: (i, 0)),
    out_shape=jax.ShapeDtypeStruct((N_Q, K), jnp.int32),
)


def _vproj_body(h_ref, wv_ref, bv_ref, v_ref):
    v_ref[...] = lax.dot_general(h_ref[...], wv_ref[...], (((1,), (0,)), ((), ())),
                                 precision=lax.Precision.HIGHEST,
                                 preferred_element_type=jnp.float32) + bv_ref[0:1, :]


RBLK_B = N_OP // 8
_vproj_call = pl.pallas_call(
    _vproj_body,
    grid=(8,),
    in_specs=[
        pl.BlockSpec((RBLK_B, LATENT), lambda i: (i, 0)),
        pl.BlockSpec((LATENT, TOTAL_DIM), lambda i: (0, 0)),
        pl.BlockSpec((8, TOTAL_DIM), lambda i: (0, 0)),
    ],
    out_specs=pl.BlockSpec((RBLK_B, TOTAL_DIM), lambda i: (i, 0)),
    out_shape=jax.ShapeDtypeStruct((N_OP, TOTAL_DIM), jnp.float32),
)


def _pos_gather_body(idx_hbm, pos128_hbm, out_hbm,
                     idx_v0, idx_v1, rows_v0, rows_v1, cmp_v, sem0, sem1):
    wid = lax.axis_index("s") * NC + lax.axis_index("c")
    lane = lax.broadcasted_iota(jnp.int32, (16,), 0)
    idx_vs = (idx_v0, idx_v1)
    rows_vs = (rows_v0, rows_v1)
    sems = (sem0, sem1)

    def prefetch(b, par):
        ebase = wid * EPW + b * EB_C
        pltpu.sync_copy(idx_hbm.at[pl.ds(ebase, EB_C)], idx_vs[par])
        pltpu.async_copy(pos128_hbm.at[idx_vs[par]], rows_vs[par], sems[par])

    def compute(b, par):
        ebase = wid * EPW + b * EB_C
        rows_v = rows_vs[par]
        pltpu.make_async_copy(pos128_hbm.at[idx_vs[par]], rows_v, sems[par]).wait()
        for g in range(EB_C // 4):
            acc = jnp.zeros((16,), jnp.float32)
            for j in range(4):
                ve = rows_v[g * 4 + j, pl.ds(0, 16)]
                acc = jnp.where(lane == j * 4, ve[0], acc)
                acc = jnp.where(lane == j * 4 + 1, ve[1], acc)
                acc = jnp.where(lane == j * 4 + 2, ve[2], acc)
            cmp_v[pl.ds(g * 16, 16)] = acc
        pltpu.sync_copy(cmp_v.at[pl.ds(0, EB_C * 4)],
                        out_hbm.at[pl.ds(ebase * 4, EB_C * 4)])

    prefetch(0, 0)

    def body(g, _):
        b0 = g * 2
        prefetch(b0 + 1, 1)
        compute(b0, 0)

        @pl.when(g < NB_C // 2 - 1)
        def _():
            prefetch(b0 + 2, 0)

        compute(b0 + 1, 1)
        return 0

    lax.fori_loop(0, NB_C // 2, body, 0)


@functools.lru_cache(maxsize=None)
def _sc_calls():
    mesh = plsc.VectorSubcoreMesh(core_axis_name="c", subcore_axis_name="s",
                                  num_cores=NC, num_subcores=NS)
    pos_gather = pl.kernel(
        _pos_gather_body,
        out_type=jax.ShapeDtypeStruct((E * 4,), jnp.float32),
        mesh=mesh,
        scratch_types=[
            pltpu.VMEM((EB_C,), jnp.int32),
            pltpu.VMEM((EB_C,), jnp.int32),
            pltpu.VMEM((EB_C, 128), jnp.float32),
            pltpu.VMEM((EB_C, 128), jnp.float32),
            pltpu.VMEM((EB_C * 4,), jnp.float32),
            pltpu.SemaphoreType.DMA,
            pltpu.SemaphoreType.DMA,
        ],
    )
    wsum = pl.kernel(
        _wsum_body,
        out_type=jax.ShapeDtypeStruct((N_Q, TOTAL_DIM), jnp.float32),
        mesh=mesh,
        scratch_types=[
            pltpu.VMEM((QB_E * K,), jnp.int32),
            pltpu.VMEM((QB_E * K,), jnp.int32),
            pltpu.VMEM((QB_E * K, TOTAL_DIM), jnp.float32),
            pltpu.VMEM((QB_E * K, TOTAL_DIM), jnp.float32),
            pltpu.VMEM((QB_E * K * HEADS + 16,), jnp.float32),
            pltpu.VMEM((QB_E, TOTAL_DIM), jnp.float32),
            pltpu.SemaphoreType.DMA,
            pltpu.SemaphoreType.DMA,
        ],
    )
    return pos_gather, wsum


QBLK_D = 128
EBLK_D = QBLK_D * K


def _mlp_body(ps_ref, qp4_ref, w1_ref, w2_ref, b2_ref, attn_ref):
    ps = ps_ref[...].reshape(EBLK_D, 4)
    a = lax.dot_general(ps, w1_ref[...], (((1,), (0,)), ((), ())),
                        precision=lax.Precision.HIGHEST,
                        preferred_element_type=jnp.float32)
    b = lax.dot_general(qp4_ref[...], w1_ref[...], (((1,), (0,)), ((), ())),
                        precision=lax.Precision.HIGHEST,
                        preferred_element_type=jnp.float32)
    bx = jnp.broadcast_to(b[:, None, :], (QBLK_D, K, LATENT)).reshape(EBLK_D, LATENT)
    h = jnp.maximum(bx - a, 0.0)
    logits = lax.dot_general(h, w2_ref[...], (((1,), (0,)), ((), ())),
                             precision=lax.Precision.HIGHEST,
                             preferred_element_type=jnp.float32) + b2_ref[0:1, 0:HEADS]
    l3 = logits.reshape(QBLK_D, K, HEADS)
    mx = jnp.max(l3, axis=1, keepdims=True)
    ex = jnp.exp(l3 - mx)
    sm = jnp.sum(ex, axis=1, keepdims=True)
    attn_ref[...] = ex / (sm + 1e-16)


_mlp_call = pl.pallas_call(
    _mlp_body,
    grid=(N_Q // QBLK_D,),
    in_specs=[
        pl.BlockSpec((QBLK_D, K, 4), lambda i: (i, 0, 0)),
        pl.BlockSpec((QBLK_D, 4), lambda i: (i, 0)),
        pl.BlockSpec((4, LATENT), lambda i: (0, 0)),
        pl.BlockSpec((LATENT, HEADS), lambda i: (0, 0)),
        pl.BlockSpec((8, 128), lambda i: (0, 0)),
    ],
    out_specs=pl.BlockSpec((QBLK_D, K, HEADS), lambda i: (i, 0, 0)),
    out_shape=jax.ShapeDtypeStruct((N_Q, K, HEADS), jnp.float32),
)


def _wsum_body(idx_hbm, attn_hbm, v_hbm, out_hbm,
               idx_v0, idx_v1, rows_v0, rows_v1, attn_v, out_v, sem0, sem1):
    wid = lax.axis_index("s") * NC + lax.axis_index("c")
    idx_vs = (idx_v0, idx_v1)
    rows_vs = (rows_v0, rows_v1)
    sems = (sem0, sem1)

    def prefetch(b, par):
        ebase = wid * EPW + b * (QB_E * K)
        pltpu.sync_copy(idx_hbm.at[pl.ds(ebase, QB_E * K)], idx_vs[par])
        pltpu.async_copy(v_hbm.at[idx_vs[par]], rows_vs[par], sems[par])

    def compute(b, par):
        ebase = wid * EPW + b * (QB_E * K)
        qbase = wid * QPW + b * QB_E
        rows_v = rows_vs[par]
        pltpu.sync_copy(attn_hbm.at[pl.ds(ebase * HEADS, QB_E * K * HEADS)],
                        attn_v.at[pl.ds(0, QB_E * K * HEADS)])
        pltpu.make_async_copy(v_hbm.at[idx_vs[par]], rows_v, sems[par]).wait()
        for qq in range(QB_E):
            def kbody(k, accs):
                new = list(accs)
                row = qq * K + k
                av = attn_v[pl.ds(qq * (K * HEADS) + k * HEADS, 16)]
                for h in range(HEADS):
                    wv = jnp.full((16,), av[h], jnp.float32)
                    for t in range(HEAD_DIM // 16):
                        j = h * 4 + t
                        new[j] = new[j] + wv * rows_v[row, pl.ds(h * HEAD_DIM + t * 16, 16)]
                return tuple(new)

            accs = lax.fori_loop(
                0, K, kbody,
                tuple(jnp.zeros((16,), jnp.float32) for _ in range(16)))
            for h in range(HEADS):
                for t in range(HEAD_DIM // 16):
                    out_v[qq, pl.ds(h * HEAD_DIM + t * 16, 16)] = accs[h * 4 + t]
        pltpu.sync_copy(out_v, out_hbm.at[pl.ds(qbase, QB_E), :])

    prefetch(0, 0)

    def body(g, _):
        b0 = g * 2
        prefetch(b0 + 1, 1)
        compute(b0, 0)

        @pl.when(g < NB_E // 2 - 1)
        def _():
            prefetch(b0 + 2, 0)

        compute(b0 + 1, 1)
        return 0

    lax.fori_loop(0, NB_E // 2, body, 0)


def kernel(h_obs, pos_obs, pos_query, W_v, b_v, W1, b1, W2, b2):
    f32 = jnp.float32
    q2 = jnp.sum(pos_query ** 2, axis=1, keepdims=True)
    o2 = jnp.sum(pos_obs ** 2, axis=1)[None, :]
    pos_obs_t = jnp.concatenate(
        [pos_obs.T, jnp.zeros((3, N_OP - N_O), f32)], axis=1)
    o2p = jnp.concatenate(
        [o2, jnp.full((1, N_OP - N_O), 1.0e30, f32)], axis=1)
    d2p = q2 + o2p - 2.0 * (pos_query @ pos_obs_t)

    h_pad = jnp.zeros((N_OP, LATENT), f32).at[0:N_O, :].set(h_obs)
    bv_p = jnp.zeros((8, TOTAL_DIM), f32).at[0, :].set(b_v)
    pos128 = jnp.zeros((N_O, 128), f32).at[:, 0:3].set(pos_obs)
    qp4 = jnp.zeros((N_Q, 4), f32).at[:, 0:3].set(pos_query)
    qp4 = qp4.at[:, 3].set(1.0)
    w1p = jnp.zeros((4, LATENT), f32).at[0:3, :].set(W1)
    w1p = w1p.at[3, :].set(b1)
    b2p = jnp.zeros((8, 128), f32).at[0, 0:HEADS].set(b2)

    pos_gather_call, wsum_call = _sc_calls()
    knn_idx = _sel_call(d2p)
    idx_flat = knn_idx.reshape(E)
    v = _vproj_call(h_pad, W_v, bv_p)
    pos_src = pos_gather_call(idx_flat, pos128)
    attn = _mlp_call(pos_src.reshape(N_Q, K, 4), qp4, w1p, W2, b2p)
    attn_flat = attn.reshape(E * HEADS)
    out = wsum_call(idx_flat, attn_flat, v)
    return out
